# Initial kernel scaffold; baseline (speedup 1.0000x reference)
import jax
import jax.numpy as jnp
from jax.experimental import pallas as pl


def kernel(u, i, neighbors, user_table, entity_table, W, a):
    raise NotImplementedError("write your pallas kernel here")


# speedup vs baseline: 1.3546x; 1.3546x over previous
import functools

import jax
import jax.numpy as jnp
from jax import lax
from jax.experimental import pallas as pl
from jax.experimental.pallas import tpu as pltpu
from jax.experimental.pallas import tpu_sc as plsc

B, K, D = 16384, 20, 16
NH = D // 2

_info = plsc.get_sparse_core_info()
NC, NS, L = _info.num_cores, _info.num_subcores, _info.num_lanes
NW = NC * NS
BPW = B // NW
RC = 128
NCHUNK = BPW // RC
NG = RC // L


def _renorm_scale(ss):
    x = jnp.maximum(ss, 1e-24)
    xi = plsc.bitcast(x, jnp.int32)
    y = plsc.bitcast(jnp.int32(0x5F3759DF) - (xi >> 1), jnp.float32)
    for _ in range(3):
        y = y * (1.5 - 0.5 * x * y * y)
    return jnp.minimum(y, 1.0)


def _body(u_hbm, i_hbm, nb_hbm, ut_hbm, et_hbm, wt_hbm, a_hbm, out_hbm,
          i_v, u_v, nb_v, t_rows, n_rows, u_rows,
          wt_v, a_v, wa0_v, wa1_v, out_v, sem):
    wid = lax.axis_index("s") * NC + lax.axis_index("c")
    base = wid * BPW

    pltpu.sync_copy(i_hbm.at[pl.ds(base, BPW)], i_v)
    pltpu.sync_copy(u_hbm.at[pl.ds(base, BPW)], u_v)
    pltpu.sync_copy(nb_hbm.at[pl.ds(base * K, BPW * K)], nb_v)
    pltpu.sync_copy(wt_hbm, wt_v)
    pltpu.sync_copy(a_hbm, a_v)

    iota = lax.iota(jnp.int32, L)

    avec = a_v[...]
    wa0 = jnp.zeros((L,), jnp.float32)
    wa1 = jnp.zeros((L,), jnp.float32)
    for j in range(NH):
        colj = wt_v[pl.ds(j * L, L)]
        wa0 = wa0 + colj * avec[j]
        wa1 = wa1 + colj * avec[NH + j]
    wa0_v[...] = wa0
    wa1_v[...] = wa1

    def chunk_body(c, carry):
        cb = c * RC
        cps = [pltpu.async_copy(et_hbm.at[i_v.at[pl.ds(cb, RC)]], t_rows, sem),
               pltpu.async_copy(ut_hbm.at[u_v.at[pl.ds(cb, RC)]], u_rows, sem)]
        for j in range(K):
            cps.append(pltpu.async_copy(
                et_hbm.at[nb_v.at[pl.ds(cb * K + j * RC, RC)]],
                n_rows.at[pl.ds(j * RC, RC)], sem))
        for cp in cps:
            cp.wait()

        def group(g, carry2):
            rows = g * L + iota
            nbase = g * (L * K)
            iota_k = iota * K
            wa0vec = wa0_v[...]
            wa1vec = wa1_v[...]
            wrows = [wt_v[pl.ds(j * L, L)] for j in range(NH)]

            ss_t = jnp.zeros((L,), jnp.float32)
            dot_t = jnp.zeros((L,), jnp.float32)
            for d in range(D):
                col = plsc.load_gather(
                    t_rows, [rows, jnp.full((L,), d, jnp.int32)])
                ss_t += col * col
                dot_t += col * wa0vec[d]
            ts = dot_t * _renorm_scale(ss_t)

            es = []
            scales = []
            for k in range(K):
                ss = jnp.zeros((L,), jnp.float32)
                dt = jnp.zeros((L,), jnp.float32)
                for d in range(D):
                    col = plsc.load_gather(
                        n_rows,
                        [iota_k + (nbase + k), jnp.full((L,), d, jnp.int32)])
                    ss += col * col
                    dt += col * wa1vec[d]
                sc = _renorm_scale(ss)
                e = ts + dt * sc
                es.append(jnp.maximum(e, 0.2 * e))
                scales.append(sc)

            m = es[0]
            for k in range(1, K):
                m = jnp.maximum(m, es[k])
            exps = [jnp.exp(e - m) for e in es]
            ssum = exps[0]
            for k in range(1, K):
                ssum = ssum + exps[k]
            inv = 1.0 / ssum
            betas = [exps[k] * inv * scales[k] for k in range(K)]

            gj = [jnp.zeros((L,), jnp.float32) for _ in range(NH)]
            for d in range(D):
                agg = jnp.zeros((L,), jnp.float32)
                for k in range(K):
                    col = plsc.load_gather(
                        n_rows,
                        [iota_k + (nbase + k), jnp.full((L,), d, jnp.int32)])
                    agg = agg + betas[k] * col
                for j in range(NH):
                    gj[j] = gj[j] + agg * wrows[j][d]

            ss_u = jnp.zeros((L,), jnp.float32)
            ucols = []
            for d in range(D):
                col = plsc.load_gather(
                    u_rows, [rows, jnp.full((L,), d, jnp.int32)])
                ss_u += col * col
                ucols.append(col)
            su = _renorm_scale(ss_u)
            uv = jnp.zeros((L,), jnp.float32)
            for j in range(NH):
                uv = uv + (ucols[j] + ucols[NH + j]) * gj[j]
            uv = uv * su
            out = 1.0 / (1.0 + jnp.exp(-uv))
            out_v[pl.ds(c * RC + g * L, L)] = out
            return carry2

        lax.fori_loop(0, NG, group, 0)
        return carry

    lax.fori_loop(0, NCHUNK, chunk_body, 0)
    pltpu.sync_copy(out_v, out_hbm.at[pl.ds(base, BPW)])


_sc_call = functools.partial(
    pl.kernel,
    out_type=jax.ShapeDtypeStruct((B,), jnp.float32),
    mesh=plsc.VectorSubcoreMesh(core_axis_name="c", subcore_axis_name="s"),
    compiler_params=pltpu.CompilerParams(needs_layout_passes=False, use_tc_tiling_on_sc=False),
    scratch_types=[
        pltpu.VMEM((BPW,), jnp.int32),
        pltpu.VMEM((BPW,), jnp.int32),
        pltpu.VMEM((BPW * K,), jnp.int32),
        pltpu.VMEM((RC, D), jnp.float32),
        pltpu.VMEM((RC * K, D), jnp.float32),
        pltpu.VMEM((RC, D), jnp.float32),
        pltpu.VMEM((NH * L,), jnp.float32),
        pltpu.VMEM((L,), jnp.float32),
        pltpu.VMEM((D,), jnp.float32),
        pltpu.VMEM((D,), jnp.float32),
        pltpu.VMEM((BPW,), jnp.float32),
        pltpu.SemaphoreType.DMA,
    ],
)(_body)


def kernel(u, i, neighbors, user_table, entity_table, W, a):
    u = u.astype(jnp.int32)
    i = i.astype(jnp.int32)
    nb = neighbors.astype(jnp.int32).reshape(-1)
    wt = W.T.reshape(-1)
    return _sc_call(u, i, nb, user_table, entity_table, wt, a.reshape(-1))
